# baseline (speedup 1.0000x reference)
import functools

import jax
import jax.numpy as jnp
from jax import lax
from jax.experimental import pallas as pl
from jax.experimental.pallas import tpu as pltpu
from jax.experimental.pallas import tpu_sc as plsc


def _vq_body(h_ref, Wq_ref, bq_ref, cb_ref, idx_ref, dsum_ref, K):
    h = h_ref[0]
    z = lax.dot_general(Wq_ref[...], h, (((1,), (0,)), ((), ())),
                        preferred_element_type=jnp.float32)
    z = z + bq_ref[...]
    cb = cb_ref[...]
    cn = jnp.sum(cb * cb, axis=1, keepdims=True)
    zn = jnp.sum(z * z, axis=0, keepdims=True)
    s2 = lax.dot_general(cb + cb, z, (((1,), (0,)), ((), ())),
                        preferred_element_type=jnp.float32)
    CH = 256
    rmin = rt = None
    for t in range(K // CH):
        st = s2[t * CH:(t + 1) * CH, :]
        cnt = cn[t * CH:(t + 1) * CH, :]
        dt = (zn + cnt) - st
        if t == 0:
            rmin = dt
            rt = jnp.zeros(dt.shape, jnp.int32)
        else:
            lt = dt < rmin
            rmin = jnp.where(lt, dt, rmin)
            rt = jnp.where(lt, t, rt)
    dmin = jnp.min(rmin, axis=0, keepdims=True)
    row = lax.broadcasted_iota(jnp.int32, rmin.shape, 0)
    kk = rt * CH + row
    idx_ref[0] = jnp.min(jnp.where(rmin == dmin, kk, K), axis=0, keepdims=True)
    dsum_ref[...] = jnp.sum(dmin, axis=1, keepdims=True)[None]


def _branch_a_kernel(h_ref, Wq_ref, bq_ref, cb_ref,
                     Wpq_a_ref, bpq_a_ref, cb_b_ref, Wpq_b_ref, bpq_b_ref,
                     idx_ref, dsum_ref, pa_ref, pb_ref, *, K):
    @pl.when(pl.program_id(0) == 0)
    def _prep():
        pa = lax.dot_general(cb_ref[...], Wpq_a_ref[...],
                             (((1,), (1,)), ((), ())),
                             preferred_element_type=jnp.float32)
        pa_ref[...] = pa + bpq_a_ref[...]
        pb = lax.dot_general(cb_b_ref[...], Wpq_b_ref[...],
                             (((1,), (1,)), ((), ())),
                             preferred_element_type=jnp.float32)
        pb_ref[...] = pb + bpq_b_ref[...]

    _vq_body(h_ref, Wq_ref, bq_ref, cb_ref, idx_ref, dsum_ref, K)


def _branch_b_kernel(h_ref, Wq_ref, bq_ref, cb_ref, idx_ref, dsum_ref, *, K):
    _vq_body(h_ref, Wq_ref, bq_ref, cb_ref, idx_ref, dsum_ref, K)


def _full(shape):
    n = len(shape)
    return pl.BlockSpec(shape, lambda b: (0,) * n)


def _vq_argmin_a(h3, Wq, bq, cb, Wpq_a, bpq_a, cb_b, Wpq_b, bpq_b):
    B, C_in, HW = h3.shape
    emb = Wq.shape[0]
    K = cb.shape[0]
    Ca = Wpq_a.shape[0]
    Cb = Wpq_b.shape[0]
    return pl.pallas_call(
        functools.partial(_branch_a_kernel, K=K),
        grid=(B,),
        in_specs=[
            pl.BlockSpec((1, C_in, HW), lambda b: (b, 0, 0)),
            _full((emb, C_in)), _full((emb, 1)), _full((K, emb)),
            _full((Ca, emb)), _full((1, Ca)),
            _full(cb_b.shape), _full(Wpq_b.shape), _full((1, Cb)),
        ],
        out_specs=[
            pl.BlockSpec((1, 1, HW), lambda b: (b, 0, 0)),
            pl.BlockSpec((1, 1, 1), lambda b: (b, 0, 0)),
            _full((K, Ca)), _full((K, Cb)),
        ],
        out_shape=[
            jax.ShapeDtypeStruct((B, 1, HW), jnp.int32),
            jax.ShapeDtypeStruct((B, 1, 1), jnp.float32),
            jax.ShapeDtypeStruct((K, Ca), jnp.float32),
            jax.ShapeDtypeStruct((K, Cb), jnp.float32),
        ],
    )(h3, Wq, bq.reshape(emb, 1), cb,
      Wpq_a, bpq_a.reshape(1, Ca), cb_b, Wpq_b, bpq_b.reshape(1, Cb))


def _vq_argmin_b(h3, Wq, bq, cb):
    B, C_in, HW = h3.shape
    emb = Wq.shape[0]
    K = cb.shape[0]
    return pl.pallas_call(
        functools.partial(_branch_b_kernel, K=K),
        grid=(B,),
        in_specs=[
            pl.BlockSpec((1, C_in, HW), lambda b: (b, 0, 0)),
            _full((emb, C_in)), _full((emb, 1)), _full((K, emb)),
        ],
        out_specs=[
            pl.BlockSpec((1, 1, HW), lambda b: (b, 0, 0)),
            pl.BlockSpec((1, 1, 1), lambda b: (b, 0, 0)),
        ],
        out_shape=[
            jax.ShapeDtypeStruct((B, 1, HW), jnp.int32),
            jax.ShapeDtypeStruct((B, 1, 1), jnp.float32),
        ],
    )(h3, Wq, bq.reshape(emb, 1), cb)


def _sc_gather_rows(table, idx_flat):
    N = idx_flat.shape[0]
    D = table.shape[1]
    info = plsc.get_sparse_core_info()
    NC = info.num_cores
    NW = NC * info.num_subcores
    n_per_w = N // NW
    CHUNK = 128
    n_chunks = n_per_w // CHUNK
    idx2 = idx_flat.reshape(N // CHUNK, CHUNK)
    mesh = plsc.VectorSubcoreMesh(core_axis_name="c", subcore_axis_name="s")

    @functools.partial(
        pl.kernel, mesh=mesh,
        out_type=jax.ShapeDtypeStruct((N, D), jnp.float32),
        scratch_types=[
            pltpu.VMEM((n_chunks, CHUNK), jnp.int32),
            pltpu.VMEM((n_per_w, D), jnp.float32),
            pltpu.SemaphoreType.DMA,
        ],
    )
    def k(table_hbm, idx_hbm, out_hbm, idx_v, rows_v, sem):
        wid = lax.axis_index("s") * NC + lax.axis_index("c")
        pltpu.sync_copy(idx_hbm.at[pl.ds(wid * n_chunks, n_chunks)], idx_v)
        copies = [
            pltpu.async_copy(table_hbm.at[idx_v.at[j]],
                             rows_v.at[pl.ds(j * CHUNK, CHUNK)], sem)
            for j in range(n_chunks)
        ]
        for c in copies:
            c.wait()
        pltpu.sync_copy(rows_v, out_hbm.at[pl.ds(wid * n_per_w, n_per_w)])

    return k(table, idx2)


def kernel(h_identity, h_others, codebook_identity, codebook_others,
           Wq_id, bq_id, Wpq_id, bpq_id, Wq_ot, bq_ot, Wpq_ot, bpq_ot):
    B, _, H, W = h_identity.shape
    HW = H * W
    emb_id = Wq_id.shape[0]
    emb_ot = Wq_ot.shape[0]

    idx_id, dsum_id, p_id, p_ot = _vq_argmin_a(
        h_identity.reshape(B, -1, HW), Wq_id, bq_id, codebook_identity,
        Wpq_id, bpq_id, codebook_others, Wpq_ot, bpq_ot)
    rows_id = _sc_gather_rows(p_id, idx_id.reshape(B * HW))
    idx_ot, dsum_ot = _vq_argmin_b(
        h_others.reshape(B, -1, HW), Wq_ot, bq_ot, codebook_others)
    rows_ot = _sc_gather_rows(p_ot, idx_ot.reshape(B * HW))

    out_id = rows_id.reshape(B, HW, -1).transpose(0, 2, 1).reshape(B, -1, H, W)
    out_ot = rows_ot.reshape(B, HW, -1).transpose(0, 2, 1).reshape(B, -1, H, W)
    loss_id = jnp.sum(dsum_id) * (1.25 / (B * HW * emb_id))
    loss_ot = jnp.sum(dsum_ot) * (1.25 / (B * HW * emb_ot))
    return (out_id, out_ot, loss_id, loss_ot,
            idx_id.reshape(B, H, W), idx_ot.reshape(B, H, W))


# speedup vs baseline: 1.0723x; 1.0723x over previous
import functools

import jax
import jax.numpy as jnp
from jax import lax
from jax.experimental import pallas as pl
from jax.experimental.pallas import tpu as pltpu
from jax.experimental.pallas import tpu_sc as plsc


def _vq_body(h_ref, Wq_ref, bq_ref, cb_ref, idx_ref, dsum_ref, K):
    h = h_ref[0]
    z = lax.dot_general(Wq_ref[...], h, (((1,), (0,)), ((), ())),
                        preferred_element_type=jnp.float32)
    z = z + bq_ref[...]
    cb = cb_ref[...]
    cn = jnp.sum(cb * cb, axis=1, keepdims=True)
    zn = jnp.sum(z * z, axis=0, keepdims=True)
    s2 = lax.dot_general(cb + cb, z, (((1,), (0,)), ((), ())),
                        preferred_element_type=jnp.float32)
    CH = 64
    rmin = rt = None
    for t in range(K // CH):
        st = s2[t * CH:(t + 1) * CH, :]
        cnt = cn[t * CH:(t + 1) * CH, :]
        dt = (zn + cnt) - st
        if t == 0:
            rmin = dt
            rt = jnp.zeros(dt.shape, jnp.int32)
        else:
            lt = dt < rmin
            rmin = jnp.where(lt, dt, rmin)
            rt = jnp.where(lt, t, rt)
    dmin = jnp.min(rmin, axis=0, keepdims=True)
    row = lax.broadcasted_iota(jnp.int32, rmin.shape, 0)
    kk = rt * CH + row
    idx_ref[0] = jnp.min(jnp.where(rmin == dmin, kk, K), axis=0, keepdims=True)
    dsum_ref[...] = jnp.sum(dmin, axis=1, keepdims=True)[None]


def _branch_a_kernel(h_ref, Wq_ref, bq_ref, cb_ref,
                     Wpq_a_ref, bpq_a_ref, cb_b_ref, Wpq_b_ref, bpq_b_ref,
                     idx_ref, dsum_ref, pa_ref, pb_ref, *, K):
    @pl.when(pl.program_id(0) == 0)
    def _prep():
        pa = lax.dot_general(cb_ref[...], Wpq_a_ref[...],
                             (((1,), (1,)), ((), ())),
                             preferred_element_type=jnp.float32)
        pa_ref[...] = pa + bpq_a_ref[...]
        pb = lax.dot_general(cb_b_ref[...], Wpq_b_ref[...],
                             (((1,), (1,)), ((), ())),
                             preferred_element_type=jnp.float32)
        pb_ref[...] = pb + bpq_b_ref[...]

    _vq_body(h_ref, Wq_ref, bq_ref, cb_ref, idx_ref, dsum_ref, K)


def _branch_b_kernel(h_ref, Wq_ref, bq_ref, cb_ref, idx_ref, dsum_ref, *, K):
    _vq_body(h_ref, Wq_ref, bq_ref, cb_ref, idx_ref, dsum_ref, K)


def _full(shape):
    n = len(shape)
    return pl.BlockSpec(shape, lambda b: (0,) * n)


def _vq_argmin_a(h3, Wq, bq, cb, Wpq_a, bpq_a, cb_b, Wpq_b, bpq_b):
    B, C_in, HW = h3.shape
    emb = Wq.shape[0]
    K = cb.shape[0]
    Ca = Wpq_a.shape[0]
    Cb = Wpq_b.shape[0]
    return pl.pallas_call(
        functools.partial(_branch_a_kernel, K=K),
        grid=(B,),
        in_specs=[
            pl.BlockSpec((1, C_in, HW), lambda b: (b, 0, 0)),
            _full((emb, C_in)), _full((emb, 1)), _full((K, emb)),
            _full((Ca, emb)), _full((1, Ca)),
            _full(cb_b.shape), _full(Wpq_b.shape), _full((1, Cb)),
        ],
        out_specs=[
            pl.BlockSpec((1, 1, HW), lambda b: (b, 0, 0)),
            pl.BlockSpec((1, 1, 1), lambda b: (b, 0, 0)),
            _full((K, Ca)), _full((K, Cb)),
        ],
        out_shape=[
            jax.ShapeDtypeStruct((B, 1, HW), jnp.int32),
            jax.ShapeDtypeStruct((B, 1, 1), jnp.float32),
            jax.ShapeDtypeStruct((K, Ca), jnp.float32),
            jax.ShapeDtypeStruct((K, Cb), jnp.float32),
        ],
    )(h3, Wq, bq.reshape(emb, 1), cb,
      Wpq_a, bpq_a.reshape(1, Ca), cb_b, Wpq_b, bpq_b.reshape(1, Cb))


def _vq_argmin_b(h3, Wq, bq, cb):
    B, C_in, HW = h3.shape
    emb = Wq.shape[0]
    K = cb.shape[0]
    return pl.pallas_call(
        functools.partial(_branch_b_kernel, K=K),
        grid=(B,),
        in_specs=[
            pl.BlockSpec((1, C_in, HW), lambda b: (b, 0, 0)),
            _full((emb, C_in)), _full((emb, 1)), _full((K, emb)),
        ],
        out_specs=[
            pl.BlockSpec((1, 1, HW), lambda b: (b, 0, 0)),
            pl.BlockSpec((1, 1, 1), lambda b: (b, 0, 0)),
        ],
        out_shape=[
            jax.ShapeDtypeStruct((B, 1, HW), jnp.int32),
            jax.ShapeDtypeStruct((B, 1, 1), jnp.float32),
        ],
    )(h3, Wq, bq.reshape(emb, 1), cb)


def _sc_gather_rows(table, idx_flat):
    N = idx_flat.shape[0]
    D = table.shape[1]
    info = plsc.get_sparse_core_info()
    NC = info.num_cores
    NW = NC * info.num_subcores
    n_per_w = N // NW
    CHUNK = 128
    n_chunks = n_per_w // CHUNK
    idx2 = idx_flat.reshape(N // CHUNK, CHUNK)
    mesh = plsc.VectorSubcoreMesh(core_axis_name="c", subcore_axis_name="s")

    @functools.partial(
        pl.kernel, mesh=mesh,
        out_type=jax.ShapeDtypeStruct((N, D), jnp.float32),
        scratch_types=[
            pltpu.VMEM((n_chunks, CHUNK), jnp.int32),
            pltpu.VMEM((n_per_w, D), jnp.float32),
            pltpu.SemaphoreType.DMA,
        ],
    )
    def k(table_hbm, idx_hbm, out_hbm, idx_v, rows_v, sem):
        wid = lax.axis_index("s") * NC + lax.axis_index("c")
        pltpu.sync_copy(idx_hbm.at[pl.ds(wid * n_chunks, n_chunks)], idx_v)
        copies = [
            pltpu.async_copy(table_hbm.at[idx_v.at[j]],
                             rows_v.at[pl.ds(j * CHUNK, CHUNK)], sem)
            for j in range(n_chunks)
        ]
        for c in copies:
            c.wait()
        pltpu.sync_copy(rows_v, out_hbm.at[pl.ds(wid * n_per_w, n_per_w)])

    return k(table, idx2)


def kernel(h_identity, h_others, codebook_identity, codebook_others,
           Wq_id, bq_id, Wpq_id, bpq_id, Wq_ot, bq_ot, Wpq_ot, bpq_ot):
    B, _, H, W = h_identity.shape
    HW = H * W
    emb_id = Wq_id.shape[0]
    emb_ot = Wq_ot.shape[0]

    idx_id, dsum_id, p_id, p_ot = _vq_argmin_a(
        h_identity.reshape(B, -1, HW), Wq_id, bq_id, codebook_identity,
        Wpq_id, bpq_id, codebook_others, Wpq_ot, bpq_ot)
    rows_id = _sc_gather_rows(p_id, idx_id.reshape(B * HW))
    idx_ot, dsum_ot = _vq_argmin_b(
        h_others.reshape(B, -1, HW), Wq_ot, bq_ot, codebook_others)
    rows_ot = _sc_gather_rows(p_ot, idx_ot.reshape(B * HW))

    out_id = rows_id.reshape(B, HW, -1).transpose(0, 2, 1).reshape(B, -1, H, W)
    out_ot = rows_ot.reshape(B, HW, -1).transpose(0, 2, 1).reshape(B, -1, H, W)
    loss_id = jnp.sum(dsum_id) * (1.25 / (B * HW * emb_id))
    loss_ot = jnp.sum(dsum_ot) * (1.25 / (B * HW * emb_ot))
    return (out_id, out_ot, loss_id, loss_ot,
            idx_id.reshape(B, H, W), idx_ot.reshape(B, H, W))


# speedup vs baseline: 1.1005x; 1.0264x over previous
import functools

import jax
import jax.numpy as jnp
from jax import lax
from jax.experimental import pallas as pl
from jax.experimental.pallas import tpu as pltpu
from jax.experimental.pallas import tpu_sc as plsc


def _vq_body(h_ref, Wq_ref, bq_ref, cb_ref, idx_ref, dsum_ref, K):
    h = h_ref[0]
    z = lax.dot_general(Wq_ref[...], h, (((1,), (0,)), ((), ())),
                        preferred_element_type=jnp.float32)
    z = z + bq_ref[...]
    cb = cb_ref[...]
    cn = jnp.sum(cb * cb, axis=1, keepdims=True)
    zn = jnp.sum(z * z, axis=0, keepdims=True)
    s2 = lax.dot_general(cb + cb, z, (((1,), (0,)), ((), ())),
                        preferred_element_type=jnp.float32)
    CH = 32
    rmin = rt = None
    for t in range(K // CH):
        st = s2[t * CH:(t + 1) * CH, :]
        cnt = cn[t * CH:(t + 1) * CH, :]
        dt = (zn + cnt) - st
        if t == 0:
            rmin = dt
            rt = jnp.zeros(dt.shape, jnp.int32)
        else:
            lt = dt < rmin
            rmin = jnp.where(lt, dt, rmin)
            rt = jnp.where(lt, t, rt)
    dmin = jnp.min(rmin, axis=0, keepdims=True)
    row = lax.broadcasted_iota(jnp.int32, rmin.shape, 0)
    kk = rt * CH + row
    idx_ref[0] = jnp.min(jnp.where(rmin == dmin, kk, K), axis=0, keepdims=True)
    dsum_ref[...] = jnp.sum(dmin, axis=1, keepdims=True)[None]


def _branch_a_kernel(h_ref, Wq_ref, bq_ref, cb_ref,
                     Wpq_a_ref, bpq_a_ref, cb_b_ref, Wpq_b_ref, bpq_b_ref,
                     idx_ref, dsum_ref, pa_ref, pb_ref, *, K):
    @pl.when(pl.program_id(0) == 0)
    def _prep():
        pa = lax.dot_general(cb_ref[...], Wpq_a_ref[...],
                             (((1,), (1,)), ((), ())),
                             preferred_element_type=jnp.float32)
        pa_ref[...] = pa + bpq_a_ref[...]
        pb = lax.dot_general(cb_b_ref[...], Wpq_b_ref[...],
                             (((1,), (1,)), ((), ())),
                             preferred_element_type=jnp.float32)
        pb_ref[...] = pb + bpq_b_ref[...]

    _vq_body(h_ref, Wq_ref, bq_ref, cb_ref, idx_ref, dsum_ref, K)


def _branch_b_kernel(h_ref, Wq_ref, bq_ref, cb_ref, idx_ref, dsum_ref, *, K):
    _vq_body(h_ref, Wq_ref, bq_ref, cb_ref, idx_ref, dsum_ref, K)


def _full(shape):
    n = len(shape)
    return pl.BlockSpec(shape, lambda b: (0,) * n)


def _vq_argmin_a(h3, Wq, bq, cb, Wpq_a, bpq_a, cb_b, Wpq_b, bpq_b):
    B, C_in, HW = h3.shape
    emb = Wq.shape[0]
    K = cb.shape[0]
    Ca = Wpq_a.shape[0]
    Cb = Wpq_b.shape[0]
    return pl.pallas_call(
        functools.partial(_branch_a_kernel, K=K),
        grid=(B,),
        in_specs=[
            pl.BlockSpec((1, C_in, HW), lambda b: (b, 0, 0)),
            _full((emb, C_in)), _full((emb, 1)), _full((K, emb)),
            _full((Ca, emb)), _full((1, Ca)),
            _full(cb_b.shape), _full(Wpq_b.shape), _full((1, Cb)),
        ],
        out_specs=[
            pl.BlockSpec((1, 1, HW), lambda b: (b, 0, 0)),
            pl.BlockSpec((1, 1, 1), lambda b: (b, 0, 0)),
            _full((K, Ca)), _full((K, Cb)),
        ],
        out_shape=[
            jax.ShapeDtypeStruct((B, 1, HW), jnp.int32),
            jax.ShapeDtypeStruct((B, 1, 1), jnp.float32),
            jax.ShapeDtypeStruct((K, Ca), jnp.float32),
            jax.ShapeDtypeStruct((K, Cb), jnp.float32),
        ],
    )(h3, Wq, bq.reshape(emb, 1), cb,
      Wpq_a, bpq_a.reshape(1, Ca), cb_b, Wpq_b, bpq_b.reshape(1, Cb))


def _vq_argmin_b(h3, Wq, bq, cb):
    B, C_in, HW = h3.shape
    emb = Wq.shape[0]
    K = cb.shape[0]
    return pl.pallas_call(
        functools.partial(_branch_b_kernel, K=K),
        grid=(B,),
        in_specs=[
            pl.BlockSpec((1, C_in, HW), lambda b: (b, 0, 0)),
            _full((emb, C_in)), _full((emb, 1)), _full((K, emb)),
        ],
        out_specs=[
            pl.BlockSpec((1, 1, HW), lambda b: (b, 0, 0)),
            pl.BlockSpec((1, 1, 1), lambda b: (b, 0, 0)),
        ],
        out_shape=[
            jax.ShapeDtypeStruct((B, 1, HW), jnp.int32),
            jax.ShapeDtypeStruct((B, 1, 1), jnp.float32),
        ],
    )(h3, Wq, bq.reshape(emb, 1), cb)


def _sc_gather_rows(table, idx_flat):
    N = idx_flat.shape[0]
    D = table.shape[1]
    info = plsc.get_sparse_core_info()
    NC = info.num_cores
    NW = NC * info.num_subcores
    n_per_w = N // NW
    CHUNK = 128
    n_chunks = n_per_w // CHUNK
    idx2 = idx_flat.reshape(N // CHUNK, CHUNK)
    mesh = plsc.VectorSubcoreMesh(core_axis_name="c", subcore_axis_name="s")

    @functools.partial(
        pl.kernel, mesh=mesh,
        out_type=jax.ShapeDtypeStruct((N, D), jnp.float32),
        scratch_types=[
            pltpu.VMEM((n_chunks, CHUNK), jnp.int32),
            pltpu.VMEM((n_per_w, D), jnp.float32),
            pltpu.SemaphoreType.DMA,
        ],
    )
    def k(table_hbm, idx_hbm, out_hbm, idx_v, rows_v, sem):
        wid = lax.axis_index("s") * NC + lax.axis_index("c")
        pltpu.sync_copy(idx_hbm.at[pl.ds(wid * n_chunks, n_chunks)], idx_v)
        copies = [
            pltpu.async_copy(table_hbm.at[idx_v.at[j]],
                             rows_v.at[pl.ds(j * CHUNK, CHUNK)], sem)
            for j in range(n_chunks)
        ]
        for c in copies:
            c.wait()
        pltpu.sync_copy(rows_v, out_hbm.at[pl.ds(wid * n_per_w, n_per_w)])

    return k(table, idx2)


def kernel(h_identity, h_others, codebook_identity, codebook_others,
           Wq_id, bq_id, Wpq_id, bpq_id, Wq_ot, bq_ot, Wpq_ot, bpq_ot):
    B, _, H, W = h_identity.shape
    HW = H * W
    emb_id = Wq_id.shape[0]
    emb_ot = Wq_ot.shape[0]

    idx_id, dsum_id, p_id, p_ot = _vq_argmin_a(
        h_identity.reshape(B, -1, HW), Wq_id, bq_id, codebook_identity,
        Wpq_id, bpq_id, codebook_others, Wpq_ot, bpq_ot)
    rows_id = _sc_gather_rows(p_id, idx_id.reshape(B * HW))
    idx_ot, dsum_ot = _vq_argmin_b(
        h_others.reshape(B, -1, HW), Wq_ot, bq_ot, codebook_others)
    rows_ot = _sc_gather_rows(p_ot, idx_ot.reshape(B * HW))

    out_id = rows_id.reshape(B, HW, -1).transpose(0, 2, 1).reshape(B, -1, H, W)
    out_ot = rows_ot.reshape(B, HW, -1).transpose(0, 2, 1).reshape(B, -1, H, W)
    loss_id = jnp.sum(dsum_id) * (1.25 / (B * HW * emb_id))
    loss_ot = jnp.sum(dsum_ot) * (1.25 / (B * HW * emb_ot))
    return (out_id, out_ot, loss_id, loss_ot,
            idx_id.reshape(B, H, W), idx_ot.reshape(B, H, W))


# speedup vs baseline: 1.1061x; 1.0050x over previous
import functools

import jax
import jax.numpy as jnp
from jax import lax
from jax.experimental import pallas as pl
from jax.experimental.pallas import tpu as pltpu
from jax.experimental.pallas import tpu_sc as plsc


def _vq_body(h_ref, Wq_ref, bq_ref, cb_ref, idx_ref, dsum_ref, K):
    h = h_ref[0]
    z = lax.dot_general(Wq_ref[...], h, (((1,), (0,)), ((), ())),
                        preferred_element_type=jnp.float32)
    z = z + bq_ref[...]
    cb = cb_ref[...]
    cn = jnp.sum(cb * cb, axis=1, keepdims=True)
    zn = jnp.sum(z * z, axis=0, keepdims=True)
    s2 = lax.dot_general(cb + cb, z, (((1,), (0,)), ((), ())),
                        preferred_element_type=jnp.float32)
    CH = 16
    rmin = rt = None
    for t in range(K // CH):
        st = s2[t * CH:(t + 1) * CH, :]
        cnt = cn[t * CH:(t + 1) * CH, :]
        dt = (zn + cnt) - st
        if t == 0:
            rmin = dt
            rt = jnp.zeros(dt.shape, jnp.int32)
        else:
            lt = dt < rmin
            rmin = jnp.where(lt, dt, rmin)
            rt = jnp.where(lt, t, rt)
    dmin = jnp.min(rmin, axis=0, keepdims=True)
    row = lax.broadcasted_iota(jnp.int32, rmin.shape, 0)
    kk = rt * CH + row
    idx_ref[0] = jnp.min(jnp.where(rmin == dmin, kk, K), axis=0, keepdims=True)
    dsum_ref[...] = jnp.sum(dmin, axis=1, keepdims=True)[None]


def _branch_a_kernel(h_ref, Wq_ref, bq_ref, cb_ref,
                     Wpq_a_ref, bpq_a_ref, cb_b_ref, Wpq_b_ref, bpq_b_ref,
                     idx_ref, dsum_ref, pa_ref, pb_ref, *, K):
    @pl.when(pl.program_id(0) == 0)
    def _prep():
        pa = lax.dot_general(cb_ref[...], Wpq_a_ref[...],
                             (((1,), (1,)), ((), ())),
                             preferred_element_type=jnp.float32)
        pa_ref[...] = pa + bpq_a_ref[...]
        pb = lax.dot_general(cb_b_ref[...], Wpq_b_ref[...],
                             (((1,), (1,)), ((), ())),
                             preferred_element_type=jnp.float32)
        pb_ref[...] = pb + bpq_b_ref[...]

    _vq_body(h_ref, Wq_ref, bq_ref, cb_ref, idx_ref, dsum_ref, K)


def _branch_b_kernel(h_ref, Wq_ref, bq_ref, cb_ref, idx_ref, dsum_ref, *, K):
    _vq_body(h_ref, Wq_ref, bq_ref, cb_ref, idx_ref, dsum_ref, K)


def _full(shape):
    n = len(shape)
    return pl.BlockSpec(shape, lambda b: (0,) * n)


def _vq_argmin_a(h3, Wq, bq, cb, Wpq_a, bpq_a, cb_b, Wpq_b, bpq_b):
    B, C_in, HW = h3.shape
    emb = Wq.shape[0]
    K = cb.shape[0]
    Ca = Wpq_a.shape[0]
    Cb = Wpq_b.shape[0]
    return pl.pallas_call(
        functools.partial(_branch_a_kernel, K=K),
        grid=(B,),
        in_specs=[
            pl.BlockSpec((1, C_in, HW), lambda b: (b, 0, 0)),
            _full((emb, C_in)), _full((emb, 1)), _full((K, emb)),
            _full((Ca, emb)), _full((1, Ca)),
            _full(cb_b.shape), _full(Wpq_b.shape), _full((1, Cb)),
        ],
        out_specs=[
            pl.BlockSpec((1, 1, HW), lambda b: (b, 0, 0)),
            pl.BlockSpec((1, 1, 1), lambda b: (b, 0, 0)),
            _full((K, Ca)), _full((K, Cb)),
        ],
        out_shape=[
            jax.ShapeDtypeStruct((B, 1, HW), jnp.int32),
            jax.ShapeDtypeStruct((B, 1, 1), jnp.float32),
            jax.ShapeDtypeStruct((K, Ca), jnp.float32),
            jax.ShapeDtypeStruct((K, Cb), jnp.float32),
        ],
    )(h3, Wq, bq.reshape(emb, 1), cb,
      Wpq_a, bpq_a.reshape(1, Ca), cb_b, Wpq_b, bpq_b.reshape(1, Cb))


def _vq_argmin_b(h3, Wq, bq, cb):
    B, C_in, HW = h3.shape
    emb = Wq.shape[0]
    K = cb.shape[0]
    return pl.pallas_call(
        functools.partial(_branch_b_kernel, K=K),
        grid=(B,),
        in_specs=[
            pl.BlockSpec((1, C_in, HW), lambda b: (b, 0, 0)),
            _full((emb, C_in)), _full((emb, 1)), _full((K, emb)),
        ],
        out_specs=[
            pl.BlockSpec((1, 1, HW), lambda b: (b, 0, 0)),
            pl.BlockSpec((1, 1, 1), lambda b: (b, 0, 0)),
        ],
        out_shape=[
            jax.ShapeDtypeStruct((B, 1, HW), jnp.int32),
            jax.ShapeDtypeStruct((B, 1, 1), jnp.float32),
        ],
    )(h3, Wq, bq.reshape(emb, 1), cb)


def _sc_gather_rows(table, idx_flat):
    N = idx_flat.shape[0]
    D = table.shape[1]
    info = plsc.get_sparse_core_info()
    NC = info.num_cores
    NW = NC * info.num_subcores
    n_per_w = N // NW
    CHUNK = 128
    n_chunks = n_per_w // CHUNK
    idx2 = idx_flat.reshape(N // CHUNK, CHUNK)
    mesh = plsc.VectorSubcoreMesh(core_axis_name="c", subcore_axis_name="s")

    @functools.partial(
        pl.kernel, mesh=mesh,
        out_type=jax.ShapeDtypeStruct((N, D), jnp.float32),
        scratch_types=[
            pltpu.VMEM((n_chunks, CHUNK), jnp.int32),
            pltpu.VMEM((n_per_w, D), jnp.float32),
            pltpu.SemaphoreType.DMA,
        ],
    )
    def k(table_hbm, idx_hbm, out_hbm, idx_v, rows_v, sem):
        wid = lax.axis_index("s") * NC + lax.axis_index("c")
        pltpu.sync_copy(idx_hbm.at[pl.ds(wid * n_chunks, n_chunks)], idx_v)
        copies = [
            pltpu.async_copy(table_hbm.at[idx_v.at[j]],
                             rows_v.at[pl.ds(j * CHUNK, CHUNK)], sem)
            for j in range(n_chunks)
        ]
        for c in copies:
            c.wait()
        pltpu.sync_copy(rows_v, out_hbm.at[pl.ds(wid * n_per_w, n_per_w)])

    return k(table, idx2)


def kernel(h_identity, h_others, codebook_identity, codebook_others,
           Wq_id, bq_id, Wpq_id, bpq_id, Wq_ot, bq_ot, Wpq_ot, bpq_ot):
    B, _, H, W = h_identity.shape
    HW = H * W
    emb_id = Wq_id.shape[0]
    emb_ot = Wq_ot.shape[0]

    idx_id, dsum_id, p_id, p_ot = _vq_argmin_a(
        h_identity.reshape(B, -1, HW), Wq_id, bq_id, codebook_identity,
        Wpq_id, bpq_id, codebook_others, Wpq_ot, bpq_ot)
    rows_id = _sc_gather_rows(p_id, idx_id.reshape(B * HW))
    idx_ot, dsum_ot = _vq_argmin_b(
        h_others.reshape(B, -1, HW), Wq_ot, bq_ot, codebook_others)
    rows_ot = _sc_gather_rows(p_ot, idx_ot.reshape(B * HW))

    out_id = rows_id.reshape(B, HW, -1).transpose(0, 2, 1).reshape(B, -1, H, W)
    out_ot = rows_ot.reshape(B, HW, -1).transpose(0, 2, 1).reshape(B, -1, H, W)
    loss_id = jnp.sum(dsum_id) * (1.25 / (B * HW * emb_id))
    loss_ot = jnp.sum(dsum_ot) * (1.25 / (B * HW * emb_ot))
    return (out_id, out_ot, loss_id, loss_ot,
            idx_id.reshape(B, H, W), idx_ot.reshape(B, H, W))
